# baseline (speedup 1.0000x reference)
import functools

import jax
import jax.numpy as jnp
from jax import lax
from jax.experimental import pallas as pl
from jax.experimental.pallas import tpu as pltpu
from jax.experimental.pallas import tpu_sc as plsc

_EMBED = 4096
_NE = 64
_K = 8
_NT = 32768
_BT = 1024
_NC = 2
_NS = 16
_NW = _NC * _NS
_TPW = _NT // _NW
_G = 16


def _tc_body(x1_ref, x2_ref, w_ref, b_ref, p_ref, pt_ref, wt_ref):
    @pl.when(pl.program_id(0) == 0)
    def _prep():
        wt_ref[...] = w_ref[...].astype(jnp.bfloat16).T

    xh1 = x1_ref[...].astype(jnp.bfloat16)
    xh2 = x2_ref[...].astype(jnp.bfloat16)
    acc = jnp.dot(xh1, wt_ref[: _EMBED // 2], preferred_element_type=jnp.float32)
    acc += jnp.dot(xh2, wt_ref[_EMBED // 2 :], preferred_element_type=jnp.float32)
    logits = acc + b_ref[...]

    lt = logits.T
    m = jnp.max(lt, axis=0, keepdims=True)
    e = jnp.exp(lt - m)
    s = jnp.sum(e, axis=0, keepdims=True)
    pt = e / s
    p_ref[...] = pt.T
    pt_ref[...] = pt


def _tc_stage(inputs, W, b):
    bb = b.reshape(1, _NE)
    return pl.pallas_call(
        _tc_body,
        grid=(_NT // _BT,),
        in_specs=[
            pl.BlockSpec((_BT, _EMBED // 2), lambda i: (i, 0)),
            pl.BlockSpec((_BT, _EMBED // 2), lambda i: (i, 1)),
            pl.BlockSpec((_NE, _EMBED), lambda i: (0, 0)),
            pl.BlockSpec((1, _NE), lambda i: (0, 0)),
        ],
        out_specs=[
            pl.BlockSpec((_BT, _NE), lambda i: (i, 0)),
            pl.BlockSpec((_NE, _BT), lambda i: (0, i)),
        ],
        out_shape=[
            jax.ShapeDtypeStruct((_NT, _NE), jnp.float32),
            jax.ShapeDtypeStruct((_NE, _NT), jnp.float32),
        ],
        scratch_shapes=[pltpu.VMEM((_EMBED, _NE), jnp.bfloat16)],
    )(inputs, inputs, W, bb)


def _sc_kernel_body(pt_hbm, idxt_hbm, pt_v, idx_v):
    wid = lax.axis_index("s") * _NC + lax.axis_index("c")
    base = wid * _TPW
    pltpu.sync_copy(pt_hbm.at[:, pl.ds(base, _TPW)], pt_v)

    def group(g, carry):
        off = g * _G
        tv = [jnp.full((16,), -1.0, jnp.float32) for _ in range(_K)]
        ti = [jnp.zeros((16,), jnp.int32) for _ in range(_K)]
        for e in range(_NE):
            b = pt_v[e, pl.ds(off, _G)]
            bi = jnp.full((16,), e, jnp.int32)
            for k in range(_K):
                gt = b > tv[k]
                tv[k], b = jnp.where(gt, b, tv[k]), jnp.where(gt, tv[k], b)
                ti[k], bi = jnp.where(gt, bi, ti[k]), jnp.where(gt, ti[k], bi)
        for k in range(_K):
            idx_v[k, pl.ds(off, _G)] = ti[k]
        return carry

    lax.fori_loop(0, _TPW // _G, group, 0)
    pltpu.sync_copy(idx_v, idxt_hbm.at[:, pl.ds(base, _TPW)])


_sc_topk = functools.partial(
    pl.kernel,
    out_type=jax.ShapeDtypeStruct((_K, _NT), jnp.int32),
    mesh=plsc.VectorSubcoreMesh(core_axis_name="c", subcore_axis_name="s"),
    scratch_types=[
        pltpu.VMEM((_NE, _TPW), jnp.float32),
        pltpu.VMEM((_K, _TPW), jnp.int32),
    ],
)(_sc_kernel_body)


@jax.jit
def kernel(inputs, W, b):
    p, pt = _tc_stage(inputs, W, b)
    idx_t = _sc_topk(pt)
    return (p, idx_t.T)


# speedup vs baseline: 1.1408x; 1.1408x over previous
import jax
import jax.numpy as jnp
from jax.experimental import pallas as pl
from jax.experimental.pallas import tpu as pltpu

_EMBED = 4096
_NE = 64
_K = 8
_NT = 32768
_BT = 1024


def _body(x1_ref, x2_ref, w_ref, b_ref, p_ref, idx_ref, wt_ref):
    @pl.when(pl.program_id(0) == 0)
    def _prep():
        wt_ref[...] = w_ref[...].astype(jnp.bfloat16).T

    xh1 = x1_ref[...].astype(jnp.bfloat16)
    xh2 = x2_ref[...].astype(jnp.bfloat16)
    acc = jnp.dot(xh1, wt_ref[: _EMBED // 2], preferred_element_type=jnp.float32)
    acc += jnp.dot(xh2, wt_ref[_EMBED // 2 :], preferred_element_type=jnp.float32)
    logits = acc + b_ref[...]

    lt = logits.T
    m = jnp.max(lt, axis=0, keepdims=True)
    e = jnp.exp(lt - m)
    s = jnp.sum(e, axis=0, keepdims=True)
    p_ref[...] = (e / s).T

    vals = lt
    iota = jax.lax.broadcasted_iota(jnp.int32, (_NE, lt.shape[1]), 0)
    rows = []
    for _ in range(_K):
        mx = jnp.max(vals, axis=0, keepdims=True)
        amin = jnp.min(jnp.where(vals >= mx, iota, _NE), axis=0, keepdims=True)
        rows.append(amin)
        vals = jnp.where(iota == amin, -jnp.inf, vals)
    idx_ref[...] = jnp.concatenate(rows, axis=0).T


@jax.jit
def kernel(inputs, W, b):
    bb = b.reshape(1, _NE)
    grid = (_NT // _BT,)
    p, idx = pl.pallas_call(
        _body,
        grid=grid,
        in_specs=[
            pl.BlockSpec((_BT, _EMBED // 2), lambda i: (i, 0)),
            pl.BlockSpec((_BT, _EMBED // 2), lambda i: (i, 1)),
            pl.BlockSpec((_NE, _EMBED), lambda i: (0, 0)),
            pl.BlockSpec((1, _NE), lambda i: (0, 0)),
        ],
        out_specs=[
            pl.BlockSpec((_BT, _NE), lambda i: (i, 0)),
            pl.BlockSpec((_BT, _K), lambda i: (i, 0)),
        ],
        out_shape=[
            jax.ShapeDtypeStruct((_NT, _NE), jnp.float32),
            jax.ShapeDtypeStruct((_NT, _K), jnp.int32),
        ],
        scratch_shapes=[pltpu.VMEM((_EMBED, _NE), jnp.bfloat16)],
    )(inputs, inputs, W, bb)
    return (p, idx)
